# baseline (speedup 1.0000x reference)
import functools

import jax
import jax.numpy as jnp
from jax import lax
from jax.experimental import pallas as pl
from jax.experimental.pallas import tpu as pltpu
from jax.experimental.pallas import tpu_sc as plsc

N = 8192
NC = 2
NS = 16
NW = NC * NS
CHUNK = N // NW
QB = 8
L = 16

_mesh = plsc.VectorSubcoreMesh(core_axis_name="c", subcore_axis_name="s")


@functools.partial(
    pl.kernel,
    mesh=_mesh,
    compiler_params=pltpu.CompilerParams(needs_layout_passes=False),
    out_type=jax.ShapeDtypeStruct((NW, L), jnp.float32),
    scratch_types=[
        pltpu.VMEM((N,), jnp.float32),
        pltpu.VMEM((N,), jnp.float32),
        pltpu.VMEM((N,), jnp.float32),
        pltpu.VMEM((N,), jnp.float32),
        pltpu.VMEM((N,), jnp.float32),
        pltpu.VMEM((N,), jnp.float32),
        pltpu.VMEM((N,), jnp.float32),
        pltpu.VMEM((N,), jnp.float32),
        pltpu.VMEM((N,), jnp.float32),
        pltpu.VMEM((N,), jnp.float32),
        pltpu.VMEM((L,), jnp.float32),
    ],
)
def _chamfer_partials(cx_hbm, cy_hbm, tx_hbm, ty_hbm,
                      cxb_hbm, cyb_hbm, txb_hbm, tyb_hbm, out_hbm,
                      cx_v, cy_v, tx_v, ty_v,
                      cxb_v, cyb_v, txb_v, tyb_v, cn_v, tn_v, row_v):
    wid = lax.axis_index("s") * NC + lax.axis_index("c")
    pltpu.sync_copy(cx_hbm, cx_v)
    pltpu.sync_copy(cy_hbm, cy_v)
    pltpu.sync_copy(tx_hbm, tx_v)
    pltpu.sync_copy(ty_hbm, ty_v)
    pltpu.sync_copy(cxb_hbm, cxb_v)
    pltpu.sync_copy(cyb_hbm, cyb_v)
    pltpu.sync_copy(txb_hbm, txb_v)
    pltpu.sync_copy(tyb_hbm, tyb_v)

    def nstep(j, carry):
        s = pl.ds(j * L, L)
        cn_v[s] = cx_v[s] * cx_v[s] + cy_v[s] * cy_v[s]
        tn_v[s] = tx_v[s] * tx_v[s] + ty_v[s] * ty_v[s]
        return carry

    lax.fori_loop(0, N // L, nstep, jnp.int32(0))

    base = wid * CHUNK

    def min_sum(qxb_v, qyb_v, qn_v, rxb_v, ryb_v, rn_v):
        def qstep(qb, acc):
            q0 = base + qb * L
            qxv = qxb_v[pl.ds(q0, L)]
            qyv = qyb_v[pl.ds(q0, L)]
            qnv = qn_v[pl.ds(q0, L)]
            for h in range(L // QB):
                qx2 = [jnp.full((L,), 2.0 * qxv[h * QB + k], jnp.float32)
                       for k in range(QB)]
                qy2 = [jnp.full((L,), 2.0 * qyv[h * QB + k], jnp.float32)
                       for k in range(QB)]
                qn = [jnp.full((L,), qnv[h * QB + k], jnp.float32)
                      for k in range(QB)]

                def jstep(j, mins):
                    sl = pl.ds(j * L, L)
                    rx = rxb_v[sl]
                    ry = ryb_v[sl]
                    rn = rn_v[sl]
                    out = []
                    for k in range(QB):
                        a = qn[k] + rn
                        b = qx2[k] * rx + qy2[k] * ry
                        s = jnp.maximum(a - b, 0.0)
                        out.append(jnp.minimum(mins[k], s))
                    return tuple(out)

                inf = jnp.full((L,), jnp.inf, jnp.float32)
                mins = lax.fori_loop(0, N // L, jstep, (inf,) * QB)
                for k in range(QB):
                    acc = acc + jnp.min(mins[k])
            return acc

        return lax.fori_loop(0, CHUNK // L, qstep, jnp.float32(0.0))

    s_c = min_sum(cxb_v, cyb_v, cn_v, txb_v, tyb_v, tn_v)
    s_t = min_sum(txb_v, tyb_v, tn_v, cxb_v, cyb_v, cn_v)

    lane = lax.iota(jnp.int32, L)
    row = jnp.where(lane == 0, jnp.full((L,), s_c, jnp.float32),
                    jnp.where(lane == 1, jnp.full((L,), s_t, jnp.float32),
                              jnp.zeros((L,), jnp.float32)))
    row_v[...] = row
    pltpu.sync_copy(row_v, out_hbm.at[wid])


def _bf16_round(x):
    u = jax.lax.bitcast_convert_type(x, jnp.uint32)
    r = u + jnp.uint32(0x7FFF) + ((u >> 16) & jnp.uint32(1))
    r = r & jnp.uint32(0xFFFF0000)
    return jax.lax.bitcast_convert_type(r, jnp.float32)


def kernel(Xc, Xt):
    cx = Xc[:, 0]
    cy = Xc[:, 1]
    tx = Xt[:, 0]
    ty = Xt[:, 1]
    cxb = _bf16_round(cx)
    cyb = _bf16_round(cy)
    txb = _bf16_round(tx)
    tyb = _bf16_round(ty)
    partials = _chamfer_partials(cx, cy, tx, ty, cxb, cyb, txb, tyb)
    return (jnp.sum(partials[:, 0]) + jnp.sum(partials[:, 1])) / N


# speedup vs baseline: 1.3489x; 1.3489x over previous
import functools

import jax
import jax.numpy as jnp
from jax import lax
from jax.experimental import pallas as pl
from jax.experimental.pallas import tpu as pltpu
from jax.experimental.pallas import tpu_sc as plsc

N = 8192
NC = 2
NS = 16
NW = NC * NS
CHUNK = N // NW
QB = 8
L = 16

_mesh = plsc.VectorSubcoreMesh(core_axis_name="c", subcore_axis_name="s")


@functools.partial(
    pl.kernel,
    mesh=_mesh,
    compiler_params=pltpu.CompilerParams(needs_layout_passes=False),
    out_type=jax.ShapeDtypeStruct((NW, L), jnp.float32),
    scratch_types=[
        pltpu.VMEM((N,), jnp.float32),
        pltpu.VMEM((N,), jnp.float32),
        pltpu.VMEM((N,), jnp.float32),
        pltpu.VMEM((N,), jnp.float32),
        pltpu.VMEM((N,), jnp.float32),
        pltpu.VMEM((N,), jnp.float32),
        pltpu.VMEM((N,), jnp.float32),
        pltpu.VMEM((N,), jnp.float32),
        pltpu.VMEM((N,), jnp.float32),
        pltpu.VMEM((N,), jnp.float32),
        pltpu.VMEM((L,), jnp.float32),
    ],
)
def _chamfer_partials(cx_hbm, cy_hbm, tx_hbm, ty_hbm,
                      cxb_hbm, cyb_hbm, txb_hbm, tyb_hbm, out_hbm,
                      cx_v, cy_v, tx_v, ty_v,
                      cxb_v, cyb_v, txb_v, tyb_v, cn_v, tn_v, row_v):
    wid = lax.axis_index("s") * NC + lax.axis_index("c")
    pltpu.sync_copy(cx_hbm, cx_v)
    pltpu.sync_copy(cy_hbm, cy_v)
    pltpu.sync_copy(tx_hbm, tx_v)
    pltpu.sync_copy(ty_hbm, ty_v)
    pltpu.sync_copy(cxb_hbm, cxb_v)
    pltpu.sync_copy(cyb_hbm, cyb_v)
    pltpu.sync_copy(txb_hbm, txb_v)
    pltpu.sync_copy(tyb_hbm, tyb_v)

    def nstep(j, carry):
        s = pl.ds(j * L, L)
        cn_v[s] = cx_v[s] * cx_v[s] + cy_v[s] * cy_v[s]
        tn_v[s] = tx_v[s] * tx_v[s] + ty_v[s] * ty_v[s]
        return carry

    lax.fori_loop(0, N // L, nstep, jnp.int32(0))

    base = wid * CHUNK

    def min_sum(qxb_v, qyb_v, qn_v, rxb_v, ryb_v, rn_v):
        def qstep(qb, acc):
            q0 = base + qb * L
            qxv = qxb_v[pl.ds(q0, L)]
            qyv = qyb_v[pl.ds(q0, L)]
            qnv = qn_v[pl.ds(q0, L)]
            for h in range(L // QB):
                qx2 = [jnp.full((L,), 2.0 * qxv[h * QB + k], jnp.float32)
                       for k in range(QB)]
                qy2 = [jnp.full((L,), 2.0 * qyv[h * QB + k], jnp.float32)
                       for k in range(QB)]

                def jstep(j, mins):
                    sl = pl.ds(j * L, L)
                    rx = rxb_v[sl]
                    ry = ryb_v[sl]
                    rn = rn_v[sl]
                    out = []
                    for k in range(QB):
                        b = qx2[k] * rx + qy2[k] * ry
                        out.append(jnp.minimum(mins[k], rn - b))
                    return tuple(out)

                inf = jnp.full((L,), jnp.inf, jnp.float32)
                mins = lax.fori_loop(0, N // L, jstep, (inf,) * QB,
                                     unroll=2)
                for k in range(QB):
                    acc = acc + jnp.maximum(qnv[h * QB + k] + jnp.min(mins[k]),
                                            0.0)
            return acc

        return lax.fori_loop(0, CHUNK // L, qstep, jnp.float32(0.0))

    s_c = min_sum(cxb_v, cyb_v, cn_v, txb_v, tyb_v, tn_v)
    s_t = min_sum(txb_v, tyb_v, tn_v, cxb_v, cyb_v, cn_v)

    lane = lax.iota(jnp.int32, L)
    row = jnp.where(lane == 0, jnp.full((L,), s_c, jnp.float32),
                    jnp.where(lane == 1, jnp.full((L,), s_t, jnp.float32),
                              jnp.zeros((L,), jnp.float32)))
    row_v[...] = row
    pltpu.sync_copy(row_v, out_hbm.at[wid])


def _bf16_round(x):
    u = jax.lax.bitcast_convert_type(x, jnp.uint32)
    r = u + jnp.uint32(0x7FFF) + ((u >> 16) & jnp.uint32(1))
    r = r & jnp.uint32(0xFFFF0000)
    return jax.lax.bitcast_convert_type(r, jnp.float32)


def kernel(Xc, Xt):
    cx = Xc[:, 0]
    cy = Xc[:, 1]
    tx = Xt[:, 0]
    ty = Xt[:, 1]
    cxb = _bf16_round(cx)
    cyb = _bf16_round(cy)
    txb = _bf16_round(tx)
    tyb = _bf16_round(ty)
    partials = _chamfer_partials(cx, cy, tx, ty, cxb, cyb, txb, tyb)
    return (jnp.sum(partials[:, 0]) + jnp.sum(partials[:, 1])) / N


# speedup vs baseline: 1.3942x; 1.0336x over previous
import functools

import jax
import jax.numpy as jnp
from jax import lax
from jax.experimental import pallas as pl
from jax.experimental.pallas import tpu as pltpu
from jax.experimental.pallas import tpu_sc as plsc

N = 8192
NC = 2
NS = 16
NW = NC * NS
CHUNK = N // NW
QB = 8
L = 16

_mesh = plsc.VectorSubcoreMesh(core_axis_name="c", subcore_axis_name="s")
_params = pltpu.CompilerParams(needs_layout_passes=False)


@functools.partial(
    pl.kernel,
    mesh=_mesh,
    compiler_params=_params,
    out_type=(jax.ShapeDtypeStruct((NW, L), jnp.float32),
              jax.ShapeDtypeStruct((NW, N), jnp.float32)),
    scratch_types=[
        pltpu.VMEM((N,), jnp.float32),
        pltpu.VMEM((N,), jnp.float32),
        pltpu.VMEM((N,), jnp.float32),
        pltpu.VMEM((N,), jnp.float32),
        pltpu.VMEM((N,), jnp.float32),
        pltpu.VMEM((N,), jnp.float32),
        pltpu.VMEM((N,), jnp.float32),
        pltpu.VMEM((N,), jnp.float32),
        pltpu.VMEM((N,), jnp.float32),
        pltpu.VMEM((N,), jnp.float32),
        pltpu.VMEM((N,), jnp.float32),
        pltpu.VMEM((L,), jnp.float32),
    ],
)
def _chamfer_main(cx_hbm, cy_hbm, tx_hbm, ty_hbm,
                  cxb_hbm, cyb_hbm, txb_hbm, tyb_hbm,
                  rowsum_hbm, colpart_hbm,
                  cx_v, cy_v, tx_v, ty_v,
                  cxb_v, cyb_v, txb_v, tyb_v, cn_v, tn_v, col_v, row_v):
    wid = lax.axis_index("s") * NC + lax.axis_index("c")
    pltpu.sync_copy(cx_hbm, cx_v)
    pltpu.sync_copy(cy_hbm, cy_v)
    pltpu.sync_copy(tx_hbm, tx_v)
    pltpu.sync_copy(ty_hbm, ty_v)
    pltpu.sync_copy(cxb_hbm, cxb_v)
    pltpu.sync_copy(cyb_hbm, cyb_v)
    pltpu.sync_copy(txb_hbm, txb_v)
    pltpu.sync_copy(tyb_hbm, tyb_v)

    inf = jnp.full((L,), jnp.inf, jnp.float32)

    def nstep(j, carry):
        sl = pl.ds(j * L, L)
        cn_v[sl] = cx_v[sl] * cx_v[sl] + cy_v[sl] * cy_v[sl]
        tn_v[sl] = tx_v[sl] * tx_v[sl] + ty_v[sl] * ty_v[sl]
        col_v[sl] = inf
        return carry

    lax.fori_loop(0, N // L, nstep, jnp.int32(0))

    base = wid * CHUNK

    def qstep(qb, acc):
        q0 = base + qb * L
        qxv = cxb_v[pl.ds(q0, L)]
        qyv = cyb_v[pl.ds(q0, L)]
        qnv = cn_v[pl.ds(q0, L)]
        for h in range(L // QB):
            qx2 = [jnp.full((L,), 2.0 * qxv[h * QB + k], jnp.float32)
                   for k in range(QB)]
            qy2 = [jnp.full((L,), 2.0 * qyv[h * QB + k], jnp.float32)
                   for k in range(QB)]
            qn = [jnp.full((L,), qnv[h * QB + k], jnp.float32)
                  for k in range(QB)]

            def jstep(j, mins):
                sl = pl.ds(j * L, L)
                rx = txb_v[sl]
                ry = tyb_v[sl]
                rn = tn_v[sl]
                cm = col_v[sl]
                out = []
                for k in range(QB):
                    b = qx2[k] * rx + qy2[k] * ry
                    out.append(jnp.minimum(mins[k], rn - b))
                    cm = jnp.minimum(cm, qn[k] - b)
                col_v[sl] = cm
                return tuple(out)

            mins = lax.fori_loop(0, N // L, jstep, (inf,) * QB, unroll=2)
            for k in range(QB):
                acc = acc + jnp.maximum(qnv[h * QB + k] + jnp.min(mins[k]),
                                        0.0)
        return acc

    s_c = lax.fori_loop(0, CHUNK // L, qstep, jnp.float32(0.0))

    lane = lax.iota(jnp.int32, L)
    row = jnp.where(lane == 0, jnp.full((L,), s_c, jnp.float32),
                    jnp.zeros((L,), jnp.float32))
    row_v[...] = row
    pltpu.sync_copy(row_v, rowsum_hbm.at[wid])
    pltpu.sync_copy(col_v, colpart_hbm.at[wid])


@functools.partial(
    pl.kernel,
    mesh=_mesh,
    compiler_params=_params,
    out_type=jax.ShapeDtypeStruct((NW, L), jnp.float32),
    scratch_types=[
        pltpu.VMEM((NW, CHUNK), jnp.float32),
        pltpu.VMEM((CHUNK,), jnp.float32),
        pltpu.VMEM((CHUNK,), jnp.float32),
        pltpu.VMEM((L,), jnp.float32),
    ],
)
def _chamfer_combine(tx_hbm, ty_hbm, colpart_hbm, out_hbm,
                     parts_v, tx_v, ty_v, row_v):
    wid = lax.axis_index("s") * NC + lax.axis_index("c")
    base = wid * CHUNK
    pltpu.sync_copy(colpart_hbm.at[:, pl.ds(base, CHUNK)], parts_v)
    pltpu.sync_copy(tx_hbm.at[pl.ds(base, CHUNK)], tx_v)
    pltpu.sync_copy(ty_hbm.at[pl.ds(base, CHUNK)], ty_v)

    def jstep(j, acc):
        sl = pl.ds(j * L, L)
        m = jnp.full((L,), jnp.inf, jnp.float32)
        for w in range(NW):
            m = jnp.minimum(m, parts_v[w, sl])
        tn = ty_v[sl] * ty_v[sl]
        tn = tx_v[sl] * tx_v[sl] + tn
        return acc + jnp.maximum(tn + m, 0.0)

    s = lax.fori_loop(0, CHUNK // L, jstep, jnp.zeros((L,), jnp.float32))
    s_t = jnp.sum(s)

    lane = lax.iota(jnp.int32, L)
    row = jnp.where(lane == 0, jnp.full((L,), s_t, jnp.float32),
                    jnp.zeros((L,), jnp.float32))
    row_v[...] = row
    pltpu.sync_copy(row_v, out_hbm.at[wid])


def _bf16_round(x):
    u = jax.lax.bitcast_convert_type(x, jnp.uint32)
    r = u + jnp.uint32(0x7FFF) + ((u >> 16) & jnp.uint32(1))
    r = r & jnp.uint32(0xFFFF0000)
    return jax.lax.bitcast_convert_type(r, jnp.float32)


def kernel(Xc, Xt):
    cx = Xc[:, 0]
    cy = Xc[:, 1]
    tx = Xt[:, 0]
    ty = Xt[:, 1]
    cxb = _bf16_round(cx)
    cyb = _bf16_round(cy)
    txb = _bf16_round(tx)
    tyb = _bf16_round(ty)
    rowsums, colparts = _chamfer_main(cx, cy, tx, ty, cxb, cyb, txb, tyb)
    colsums = _chamfer_combine(tx, ty, colparts)
    return (jnp.sum(rowsums[:, 0]) + jnp.sum(colsums[:, 0])) / N
